# baseline (speedup 1.0000x reference)
import functools

import jax
import jax.numpy as jnp
from jax import lax
from jax.experimental import pallas as pl
from jax.experimental.pallas import tpu as pltpu
from jax.experimental.pallas import tpu_sc as plsc

B, H, W, C = 8, 224, 224, 96
CP = 128
N = B * H * W
NW = 32
CHUNK = 112
NCHUNK = N // (NW * CHUNK)
LG = CHUNK // 16


def _floor(x):
    t = x.astype(jnp.int32)
    return jnp.where(x < t.astype(jnp.float32), t - 1, t)


def _prep_body(in_ref, out_ref):
    x = in_ref[0, 0]
    xp = jnp.pad(x, ((0, 32), (0, 32)))
    xt = xp.T
    out_ref[...] = xt[:224, :]


def _tc_prep(imT):
    return pl.pallas_call(
        _prep_body,
        grid=(B, H),
        in_specs=[pl.BlockSpec((1, 1, C, W), lambda b, h: (b, h, 0, 0))],
        out_specs=pl.BlockSpec((W, CP), lambda b, h: (b * H + h, 0)),
        out_shape=jax.ShapeDtypeStruct((N, CP), jnp.float32),
    )(imT)


def _post_body(in_ref, out_ref):
    x = in_ref[...]
    xp = jnp.pad(x, ((0, 32), (0, 0)))
    xt = xp.T
    out_ref[0, 0] = xt[:C, :W]


def _tc_post(out128):
    return pl.pallas_call(
        _post_body,
        grid=(B, H),
        in_specs=[pl.BlockSpec((W, CP), lambda b, h: (b * H + h, 0))],
        out_specs=pl.BlockSpec((1, 1, C, W), lambda b, h: (b, h, 0, 0)),
        out_shape=jax.ShapeDtypeStruct((B, H, C, W), jnp.float32),
    )(out128)


def _sc_body(img_hbm, dx_hbm, dy_hbm, out_hbm,
             dxv, dyv, ia, ib, ic, id_, wav, wbv, wcv, wdv,
             bufa, bufb, bufc, bufd, outv, sem):
    wid = lax.axis_index("s") * 2 + lax.axis_index("c")
    b = lax.shift_right_logical(wid, 2)
    r0 = (wid & 3) * (NCHUNK // 2)
    bbase = b * (H * W)

    lanes = lax.iota(jnp.int32, 16)

    def initpad(k, carry):
        zero = jnp.zeros((16,), jnp.float32)
        outv[k, pl.ds(96, 16)] = zero
        outv[k, pl.ds(112, 16)] = zero
        return carry

    lax.fori_loop(0, CHUNK, initpad, 0)

    def chunk(t, carry):
        r = r0 + lax.shift_right_logical(t, 1)
        cb = (t & 1) * CHUNK
        p0 = bbase + r * W + cb
        pltpu.sync_copy(dx_hbm.at[pl.ds(p0, CHUNK)], dxv)
        pltpu.sync_copy(dy_hbm.at[pl.ds(p0, CHUNK)], dyv)

        yb = r.astype(jnp.float32) + 1.0
        xb = cb.astype(jnp.float32) + 1.0

        for i in range(LG):
            s = pl.ds(i * 16, 16)
            x = dxv[s] + ((lanes + i * 16).astype(jnp.float32) + xb)
            y = dyv[s] + yb
            x = jnp.clip(x, -8.0, 232.0)
            y = jnp.clip(y, -8.0, 232.0)
            xf = _floor(x)
            yf = _floor(y)
            x0 = jnp.clip(xf, 0, W + 1)
            x1 = jnp.clip(xf + 1, 0, W + 1)
            y0 = jnp.clip(yf, 0, H + 1)
            y1 = jnp.clip(yf + 1, 0, H + 1)
            dxw = x1.astype(jnp.float32) - x
            dyw = y1.astype(jnp.float32) - y
            vx0 = (x0 >= 1) & (x0 <= W)
            vx1 = (x1 >= 1) & (x1 <= W)
            vy0 = (y0 >= 1) & (y0 <= H)
            vy1 = (y1 >= 1) & (y1 <= H)
            zero = jnp.zeros((16,), jnp.float32)
            wa = jnp.where(vx0 & vy0, dxw * dyw, zero)
            wb = jnp.where(vx0 & vy1, dxw * (1.0 - dyw), zero)
            wc = jnp.where(vx1 & vy0, (1.0 - dxw) * dyw, zero)
            wd = jnp.where(vx1 & vy1, (1.0 - dxw) * (1.0 - dyw), zero)
            xu0 = jnp.clip(x0 - 1, 0, W - 1)
            xu1 = jnp.clip(x1 - 1, 0, W - 1)
            yu0 = jnp.clip(y0 - 1, 0, H - 1) * W + bbase
            yu1 = jnp.clip(y1 - 1, 0, H - 1) * W + bbase
            ia[s] = yu0 + xu0
            ib[s] = yu1 + xu0
            ic[s] = yu0 + xu1
            id_[s] = yu1 + xu1
            wav[s] = wa
            wbv[s] = wb
            wcv[s] = wc
            wdv[s] = wd

        copies = []
        for idx, buf in ((ia, bufa), (ib, bufb), (ic, bufc), (id_, bufd)):
            copies.append(pltpu.async_copy(img_hbm.at[idx], buf, sem))
        for cp in copies:
            cp.wait()

        def combine(i, carry2):
            s16 = pl.ds(i * 16, 16)
            wa16 = wav[s16]
            wb16 = wbv[s16]
            wc16 = wcv[s16]
            wd16 = wdv[s16]
            k0 = i * 16
            for k2 in range(16):
                k = k0 + k2
                wa = wa16[k2]
                wb = wb16[k2]
                wc = wc16[k2]
                wd = wd16[k2]
                for jj in range(C // 16):
                    s = pl.ds(jj * 16, 16)
                    outv[k, s] = (wa * bufa[k, s] + wb * bufb[k, s]
                                  + wc * bufc[k, s] + wd * bufd[k, s])
            return carry2

        lax.fori_loop(0, LG, combine, 0)
        pltpu.sync_copy(outv, out_hbm.at[pl.ds(p0, CHUNK), :])
        return carry

    lax.fori_loop(0, NCHUNK, chunk, 0)


@jax.jit
def _run(img128, dx, dy):
    kern = functools.partial(
        pl.kernel,
        mesh=plsc.VectorSubcoreMesh(core_axis_name="c", subcore_axis_name="s"),
        out_type=jax.ShapeDtypeStruct((N, CP), jnp.float32),
        scratch_types=[
            pltpu.VMEM((CHUNK,), jnp.float32),
            pltpu.VMEM((CHUNK,), jnp.float32),
            pltpu.VMEM((CHUNK,), jnp.int32),
            pltpu.VMEM((CHUNK,), jnp.int32),
            pltpu.VMEM((CHUNK,), jnp.int32),
            pltpu.VMEM((CHUNK,), jnp.int32),
            pltpu.VMEM((CHUNK,), jnp.float32),
            pltpu.VMEM((CHUNK,), jnp.float32),
            pltpu.VMEM((CHUNK,), jnp.float32),
            pltpu.VMEM((CHUNK,), jnp.float32),
            pltpu.VMEM((CHUNK, CP), jnp.float32),
            pltpu.VMEM((CHUNK, CP), jnp.float32),
            pltpu.VMEM((CHUNK, CP), jnp.float32),
            pltpu.VMEM((CHUNK, CP), jnp.float32),
            pltpu.VMEM((CHUNK, CP), jnp.float32),
            pltpu.SemaphoreType.DMA,
        ],
        compiler_params=pltpu.CompilerParams(use_tc_tiling_on_sc=False),
    )(_sc_body)
    return kern(img128, dx, dy)


def kernel(image, offsets):
    imT = jnp.transpose(image, (0, 1, 3, 2))
    img128 = _tc_prep(imT)
    dx = offsets[..., 0].reshape(N)
    dy = offsets[..., 1].reshape(N)
    out128 = _run(img128, dx, dy)
    outT = _tc_post(out128)
    return jnp.transpose(outT, (0, 1, 3, 2))


# speedup vs baseline: 2.2148x; 2.2148x over previous
import functools

import jax
import jax.numpy as jnp
from jax import lax
from jax.experimental import pallas as pl
from jax.experimental.pallas import tpu as pltpu
from jax.experimental.pallas import tpu_sc as plsc

B, H, W, C = 8, 224, 224, 96
CP = 128
N = B * H * W
NW = 32
CHUNK = 112
NCHUNK = N // (NW * CHUNK)
LG = CHUNK // 16


def _floor(x):
    t = x.astype(jnp.int32)
    return jnp.where(x < t.astype(jnp.float32), t - 1, t)


ROWS = 8


def _prep_body(in_ref, out_ref):
    for h2 in range(ROWS):
        x = in_ref[0, h2]
        xp = jnp.pad(x, ((0, 32), (0, 32)))
        xt = xp.T
        out_ref[pl.ds(h2 * W, W), :] = xt[:W, :]


def _tc_prep(imT):
    return pl.pallas_call(
        _prep_body,
        grid=(B, H // ROWS),
        in_specs=[pl.BlockSpec((1, ROWS, C, W), lambda b, h: (b, h, 0, 0))],
        out_specs=pl.BlockSpec((ROWS * W, CP), lambda b, h: (b * (H // ROWS) + h, 0)),
        out_shape=jax.ShapeDtypeStruct((N, CP), jnp.float32),
    )(imT)


def _post_body(in_ref, out_ref):
    for h2 in range(ROWS):
        x = in_ref[pl.ds(h2 * W, W), :]
        xp = jnp.pad(x, ((0, 32), (0, 0)))
        xt = xp.T
        out_ref[0, h2] = xt[:C, :W]


def _tc_post(out128):
    return pl.pallas_call(
        _post_body,
        grid=(B, H // ROWS),
        in_specs=[pl.BlockSpec((ROWS * W, CP), lambda b, h: (b * (H // ROWS) + h, 0))],
        out_specs=pl.BlockSpec((1, ROWS, C, W), lambda b, h: (b, h, 0, 0)),
        out_shape=jax.ShapeDtypeStruct((B, H, C, W), jnp.float32),
    )(out128)


def _sc_body(img_hbm, dx_hbm, dy_hbm, out_hbm,
             dxv, dyv, ia, ib, ic, id_, wav, wbv, wcv, wdv,
             bufa, bufb, bufc, bufd, outv, sem):
    wid = lax.axis_index("s") * 2 + lax.axis_index("c")
    b = lax.shift_right_logical(wid, 2)
    r0 = (wid & 3) * (NCHUNK // 2)
    bbase = b * (H * W)

    lanes = lax.iota(jnp.int32, 16)

    def initpad(k, carry):
        zero = jnp.zeros((16,), jnp.float32)
        outv[k, pl.ds(96, 16)] = zero
        outv[k, pl.ds(112, 16)] = zero
        return carry

    lax.fori_loop(0, CHUNK, initpad, 0)

    def chunk(t, carry):
        r = r0 + lax.shift_right_logical(t, 1)
        cb = (t & 1) * CHUNK
        p0 = bbase + r * W + cb
        pltpu.sync_copy(dx_hbm.at[pl.ds(p0, CHUNK)], dxv)
        pltpu.sync_copy(dy_hbm.at[pl.ds(p0, CHUNK)], dyv)

        yb = r.astype(jnp.float32) + 1.0
        xb = cb.astype(jnp.float32) + 1.0

        for i in range(LG):
            s = pl.ds(i * 16, 16)
            x = dxv[s] + ((lanes + i * 16).astype(jnp.float32) + xb)
            y = dyv[s] + yb
            x = jnp.clip(x, -8.0, 232.0)
            y = jnp.clip(y, -8.0, 232.0)
            xf = _floor(x)
            yf = _floor(y)
            x0 = jnp.clip(xf, 0, W + 1)
            x1 = jnp.clip(xf + 1, 0, W + 1)
            y0 = jnp.clip(yf, 0, H + 1)
            y1 = jnp.clip(yf + 1, 0, H + 1)
            dxw = x1.astype(jnp.float32) - x
            dyw = y1.astype(jnp.float32) - y
            vx0 = (x0 >= 1) & (x0 <= W)
            vx1 = (x1 >= 1) & (x1 <= W)
            vy0 = (y0 >= 1) & (y0 <= H)
            vy1 = (y1 >= 1) & (y1 <= H)
            zero = jnp.zeros((16,), jnp.float32)
            wa = jnp.where(vx0 & vy0, dxw * dyw, zero)
            wb = jnp.where(vx0 & vy1, dxw * (1.0 - dyw), zero)
            wc = jnp.where(vx1 & vy0, (1.0 - dxw) * dyw, zero)
            wd = jnp.where(vx1 & vy1, (1.0 - dxw) * (1.0 - dyw), zero)
            xu0 = jnp.clip(x0 - 1, 0, W - 1)
            xu1 = jnp.clip(x1 - 1, 0, W - 1)
            yu0 = jnp.clip(y0 - 1, 0, H - 1) * W + bbase
            yu1 = jnp.clip(y1 - 1, 0, H - 1) * W + bbase
            ia[s] = yu0 + xu0
            ib[s] = yu1 + xu0
            ic[s] = yu0 + xu1
            id_[s] = yu1 + xu1
            wav[s] = wa
            wbv[s] = wb
            wcv[s] = wc
            wdv[s] = wd

        copies = []
        for idx, buf in ((ia, bufa), (ib, bufb), (ic, bufc), (id_, bufd)):
            copies.append(pltpu.async_copy(img_hbm.at[idx], buf, sem))
        for cp in copies:
            cp.wait()

        def combine(i, carry2):
            s16 = pl.ds(i * 16, 16)
            wa16 = wav[s16]
            wb16 = wbv[s16]
            wc16 = wcv[s16]
            wd16 = wdv[s16]
            k0 = i * 16
            for k2 in range(16):
                k = k0 + k2
                wa = wa16[k2]
                wb = wb16[k2]
                wc = wc16[k2]
                wd = wd16[k2]
                for jj in range(C // 16):
                    s = pl.ds(jj * 16, 16)
                    outv[k, s] = (wa * bufa[k, s] + wb * bufb[k, s]
                                  + wc * bufc[k, s] + wd * bufd[k, s])
            return carry2

        lax.fori_loop(0, LG, combine, 0)
        pltpu.sync_copy(outv, out_hbm.at[pl.ds(p0, CHUNK), :])
        return carry

    lax.fori_loop(0, NCHUNK, chunk, 0)


@jax.jit
def _run(img128, dx, dy):
    kern = functools.partial(
        pl.kernel,
        mesh=plsc.VectorSubcoreMesh(core_axis_name="c", subcore_axis_name="s"),
        out_type=jax.ShapeDtypeStruct((N, CP), jnp.float32),
        scratch_types=[
            pltpu.VMEM((CHUNK,), jnp.float32),
            pltpu.VMEM((CHUNK,), jnp.float32),
            pltpu.VMEM((CHUNK,), jnp.int32),
            pltpu.VMEM((CHUNK,), jnp.int32),
            pltpu.VMEM((CHUNK,), jnp.int32),
            pltpu.VMEM((CHUNK,), jnp.int32),
            pltpu.VMEM((CHUNK,), jnp.float32),
            pltpu.VMEM((CHUNK,), jnp.float32),
            pltpu.VMEM((CHUNK,), jnp.float32),
            pltpu.VMEM((CHUNK,), jnp.float32),
            pltpu.VMEM((CHUNK, CP), jnp.float32),
            pltpu.VMEM((CHUNK, CP), jnp.float32),
            pltpu.VMEM((CHUNK, CP), jnp.float32),
            pltpu.VMEM((CHUNK, CP), jnp.float32),
            pltpu.VMEM((CHUNK, CP), jnp.float32),
            pltpu.SemaphoreType.DMA,
        ],
        compiler_params=pltpu.CompilerParams(use_tc_tiling_on_sc=False),
    )(_sc_body)
    return kern(img128, dx, dy)


def kernel(image, offsets):
    imT = jnp.transpose(image, (0, 1, 3, 2))
    img128 = _tc_prep(imT)
    dx = offsets[..., 0].reshape(N)
    dy = offsets[..., 1].reshape(N)
    out128 = _run(img128, dx, dy)
    outT = _tc_post(out128)
    return jnp.transpose(outT, (0, 1, 3, 2))


# speedup vs baseline: 2.3727x; 1.0713x over previous
import functools

import jax
import jax.numpy as jnp
from jax import lax
from jax.experimental import pallas as pl
from jax.experimental.pallas import tpu as pltpu
from jax.experimental.pallas import tpu_sc as plsc

B, H, W, C = 8, 224, 224, 96
CP = 128
N = B * H * W
NW = 32
CHUNK = 112
NCHUNK = N // (NW * CHUNK)
LG = CHUNK // 16


def _floor(x):
    t = x.astype(jnp.int32)
    return jnp.where(x < t.astype(jnp.float32), t - 1, t)


ROWS = 8


def _prep_body(in_ref, out_ref):
    for h2 in range(ROWS):
        x = in_ref[0, h2]
        xp = jnp.pad(x, ((0, 32), (0, 32)))
        xt = xp.T
        out_ref[pl.ds(h2 * W, W), :] = xt[:W, :]


def _tc_prep(imT):
    return pl.pallas_call(
        _prep_body,
        grid=(B, H // ROWS),
        in_specs=[pl.BlockSpec((1, ROWS, C, W), lambda b, h: (b, h, 0, 0))],
        out_specs=pl.BlockSpec((ROWS * W, CP), lambda b, h: (b * (H // ROWS) + h, 0)),
        out_shape=jax.ShapeDtypeStruct((N, CP), jnp.float32),
    )(imT)


def _post_body(in_ref, out_ref):
    for h2 in range(ROWS):
        x = in_ref[pl.ds(h2 * W, W), :]
        xp = jnp.pad(x, ((0, 32), (0, 0)))
        xt = xp.T
        out_ref[0, h2] = xt[:C, :W]


def _tc_post(out128):
    return pl.pallas_call(
        _post_body,
        grid=(B, H // ROWS),
        in_specs=[pl.BlockSpec((ROWS * W, CP), lambda b, h: (b * (H // ROWS) + h, 0))],
        out_specs=pl.BlockSpec((1, ROWS, C, W), lambda b, h: (b, h, 0, 0)),
        out_shape=jax.ShapeDtypeStruct((B, H, C, W), jnp.float32),
    )(out128)


def _sc_body(img_hbm, dx_hbm, dy_hbm, out_hbm,
             dxv, dyv, idx4, wv4, bufs, outv, semG, semO, semS):
    wid = lax.axis_index("s") * 2 + lax.axis_index("c")
    b = lax.shift_right_logical(wid, 2)
    r0 = (wid & 3) * (NCHUNK // 2)
    bbase = b * (H * W)

    lanes = lax.iota(jnp.int32, 16)

    def p0_of(t):
        r = r0 + lax.shift_right_logical(t, 1)
        return bbase + r * W + (t & 1) * CHUNK, r

    def fire_offsets(t, S):
        @pl.when(t < NCHUNK)
        def _():
            p0, _ = p0_of(t)
            pltpu.async_copy(dx_hbm.at[pl.ds(p0, CHUNK)], dxv.at[S], semO[S])
            pltpu.async_copy(dy_hbm.at[pl.ds(p0, CHUNK)], dyv.at[S], semO[S])

    def stage(t, S):
        p0, r = p0_of(t)
        pltpu.make_async_copy(dx_hbm.at[pl.ds(p0, CHUNK)], dxv.at[S], semO[S]).wait()
        pltpu.make_async_copy(dy_hbm.at[pl.ds(p0, CHUNK)], dyv.at[S], semO[S]).wait()

        yb = r.astype(jnp.float32) + 1.0
        xb = float(S * CHUNK) + 1.0

        for i in range(LG):
            s = pl.ds(i * 16, 16)
            x = dxv[S, s] + ((lanes + i * 16).astype(jnp.float32) + xb)
            y = dyv[S, s] + yb
            x = jnp.clip(x, -8.0, 232.0)
            y = jnp.clip(y, -8.0, 232.0)
            xf = _floor(x)
            yf = _floor(y)
            x0 = jnp.clip(xf, 0, W + 1)
            x1 = jnp.clip(xf + 1, 0, W + 1)
            y0 = jnp.clip(yf, 0, H + 1)
            y1 = jnp.clip(yf + 1, 0, H + 1)
            dxw = x1.astype(jnp.float32) - x
            dyw = y1.astype(jnp.float32) - y
            vx0 = (x0 >= 1) & (x0 <= W)
            vx1 = (x1 >= 1) & (x1 <= W)
            vy0 = (y0 >= 1) & (y0 <= H)
            vy1 = (y1 >= 1) & (y1 <= H)
            zero = jnp.zeros((16,), jnp.float32)
            wa = jnp.where(vx0 & vy0, dxw * dyw, zero)
            wb = jnp.where(vx0 & vy1, dxw * (1.0 - dyw), zero)
            wc = jnp.where(vx1 & vy0, (1.0 - dxw) * dyw, zero)
            wd = jnp.where(vx1 & vy1, (1.0 - dxw) * (1.0 - dyw), zero)
            xu0 = jnp.clip(x0 - 1, 0, W - 1)
            xu1 = jnp.clip(x1 - 1, 0, W - 1)
            yu0 = jnp.clip(y0 - 1, 0, H - 1) * W + bbase
            yu1 = jnp.clip(y1 - 1, 0, H - 1) * W + bbase
            idx4[S][0][s] = yu0 + xu0
            idx4[S][1][s] = yu1 + xu0
            idx4[S][2][s] = yu0 + xu1
            idx4[S][3][s] = yu1 + xu1
            wv4[S][0][s] = wa
            wv4[S][1][s] = wb
            wv4[S][2][s] = wc
            wv4[S][3][s] = wd

        fire_offsets(t + 2, S)
        for q in range(4):
            pltpu.async_copy(img_hbm.at[idx4[S][q]], bufs[S][q], semG[S])

    def wait_gathers(S):
        for q in range(4):
            pltpu.make_async_copy(img_hbm.at[idx4[S][q]], bufs[S][q], semG[S]).wait()

    def combine(S):
        ba, bb, bc, bd = bufs[S]
        wva, wvb, wvc, wvd = wv4[S]

        def body(i, carry2):
            s16 = pl.ds(i * 16, 16)
            wa16 = wva[s16]
            wb16 = wvb[s16]
            wc16 = wvc[s16]
            wd16 = wvd[s16]
            k0 = i * 16
            for k2 in range(16):
                k = k0 + k2
                wa = wa16[k2]
                wb = wb16[k2]
                wc = wc16[k2]
                wd = wd16[k2]
                for jj in range(C // 16):
                    s = pl.ds(jj * 16, 16)
                    outv[k, s] = (wa * ba[k, s] + wb * bb[k, s]
                                  + wc * bc[k, s] + wd * bd[k, s])
            return carry2

        lax.fori_loop(0, LG, body, 0)

    def store(t):
        p0, _ = p0_of(t)
        pltpu.async_copy(outv, out_hbm.at[pl.ds(p0, CHUNK), pl.ds(0, C)], semS)

    def drain_store(t):
        p0, _ = p0_of(t)
        pltpu.make_async_copy(outv, out_hbm.at[pl.ds(p0, CHUNK), pl.ds(0, C)], semS).wait()

    fire_offsets(0, 0)
    fire_offsets(1, 1)
    stage(0, 0)

    def body(t2, carry):
        t = 2 * t2
        stage(t + 1, 1)
        wait_gathers(0)

        @pl.when(t2 > 0)
        def _():
            drain_store(t - 1)

        combine(0)
        store(t)

        @pl.when(t2 < NCHUNK // 2 - 1)
        def _():
            stage(t + 2, 0)

        wait_gathers(1)
        drain_store(t)
        combine(1)
        store(t + 1)
        return carry

    lax.fori_loop(0, NCHUNK // 2, body, 0)
    drain_store(NCHUNK - 1)


@jax.jit
def _run(img128, dx, dy):
    kern = functools.partial(
        pl.kernel,
        mesh=plsc.VectorSubcoreMesh(core_axis_name="c", subcore_axis_name="s"),
        out_type=jax.ShapeDtypeStruct((N, CP), jnp.float32),
        scratch_types=[
            pltpu.VMEM((2, CHUNK), jnp.float32),
            pltpu.VMEM((2, CHUNK), jnp.float32),
            [[pltpu.VMEM((CHUNK,), jnp.int32) for _ in range(4)]
             for _ in range(2)],
            [[pltpu.VMEM((CHUNK,), jnp.float32) for _ in range(4)]
             for _ in range(2)],
            [[pltpu.VMEM((CHUNK, CP), jnp.float32) for _ in range(4)]
             for _ in range(2)],
            pltpu.VMEM((CHUNK, C), jnp.float32),
            [pltpu.SemaphoreType.DMA for _ in range(2)],
            [pltpu.SemaphoreType.DMA for _ in range(2)],
            pltpu.SemaphoreType.DMA,
        ],
        compiler_params=pltpu.CompilerParams(use_tc_tiling_on_sc=False),
    )(_sc_body)
    return kern(img128, dx, dy)


def kernel(image, offsets):
    imT = jnp.transpose(image, (0, 1, 3, 2))
    img128 = _tc_prep(imT)
    dx = offsets[..., 0].reshape(N)
    dy = offsets[..., 1].reshape(N)
    out128 = _run(img128, dx, dy)
    outT = _tc_post(out128)
    return jnp.transpose(outT, (0, 1, 3, 2))


# speedup vs baseline: 2.5937x; 1.0932x over previous
import functools

import jax
import jax.numpy as jnp
from jax import lax
from jax.experimental import pallas as pl
from jax.experimental.pallas import tpu as pltpu
from jax.experimental.pallas import tpu_sc as plsc

B, H, W, C = 8, 224, 224, 96
CP = 128
HW = H * W
N = B * HW
NQ = 4
BQ = B // NQ
NB = BQ * HW
NW = 32
CHUNK = 112
NCHUNK = NB // (NW * CHUNK)
WPB = NW // BQ
RPW = H // WPB
LG = CHUNK // 16
ROWS = 8
HB = H // ROWS


def _floor(x):
    t = x.astype(jnp.int32)
    return jnp.where(x < t.astype(jnp.float32), t - 1, t)


def _prep_body(in_ref, out_ref):
    for h2 in range(ROWS):
        x = in_ref[0, h2]
        xp = jnp.pad(x, ((0, 32), (0, 32)))
        xt = xp.T
        out_ref[pl.ds(h2 * W, W), :] = xt[:W, :]


def _tc_prep(imT, q):
    return pl.pallas_call(
        _prep_body,
        grid=(BQ, HB),
        in_specs=[pl.BlockSpec((1, ROWS, C, W),
                               lambda b, h: (q * BQ + b, h, 0, 0))],
        out_specs=pl.BlockSpec((ROWS * W, CP), lambda b, h: (b * HB + h, 0)),
        out_shape=jax.ShapeDtypeStruct((NB, CP), jnp.float32),
    )(imT)


def _post_body(i0, i1, i2, i3, out_ref):
    qid = pl.program_id(0) // BQ
    for h2 in range(ROWS):
        s = pl.ds(h2 * W, W)
        x = jnp.where(qid == 0, i0[s, :],
                      jnp.where(qid == 1, i1[s, :],
                                jnp.where(qid == 2, i2[s, :], i3[s, :])))
        xp = jnp.pad(x, ((0, 32), (0, 0)))
        xt = xp.T
        out_ref[0, h2] = xt[:C, :W]


def _in_spec(k):
    def imap(b, h):
        return (jnp.where(b // BQ == k, (b % BQ) * HB + h, 0), 0)
    return pl.BlockSpec((ROWS * W, CP), imap)


def _tc_post(outs):
    return pl.pallas_call(
        _post_body,
        grid=(B, HB),
        in_specs=[_in_spec(k) for k in range(NQ)],
        out_specs=pl.BlockSpec((1, ROWS, C, W), lambda b, h: (b, h, 0, 0)),
        out_shape=jax.ShapeDtypeStruct((B, H, C, W), jnp.float32),
    )(*outs)


def _make_sc_body(qoff):
    def _sc_body(img_hbm, dx_hbm, dy_hbm, out_hbm,
                 dxv, dyv, idx4, wv4, bufs, outv, semG, semO, semS):
        wid = lax.axis_index("s") * 2 + lax.axis_index("c")
        b = lax.shift_right_logical(wid, 4)
        r0 = (wid & (WPB - 1)) * RPW
        bbase = b * HW

        lanes = lax.iota(jnp.int32, 16)

        def p0_of(t):
            r = r0 + lax.shift_right_logical(t, 1)
            return bbase + r * W + (t & 1) * CHUNK, r

        def fire_offsets(t, S):
            @pl.when(t < NCHUNK)
            def _():
                p0, _ = p0_of(t)
                pltpu.async_copy(dx_hbm.at[pl.ds(p0 + qoff, CHUNK)], dxv.at[S], semO[S])
                pltpu.async_copy(dy_hbm.at[pl.ds(p0 + qoff, CHUNK)], dyv.at[S], semO[S])

        def stage(t, S):
            p0, r = p0_of(t)
            pltpu.make_async_copy(dx_hbm.at[pl.ds(p0 + qoff, CHUNK)], dxv.at[S], semO[S]).wait()
            pltpu.make_async_copy(dy_hbm.at[pl.ds(p0 + qoff, CHUNK)], dyv.at[S], semO[S]).wait()

            yb = r.astype(jnp.float32) + 1.0
            xb = float(S * CHUNK) + 1.0

            for i in range(LG):
                s = pl.ds(i * 16, 16)
                x = dxv[S, s] + ((lanes + i * 16).astype(jnp.float32) + xb)
                y = dyv[S, s] + yb
                x = jnp.clip(x, -8.0, 232.0)
                y = jnp.clip(y, -8.0, 232.0)
                xf = _floor(x)
                yf = _floor(y)
                x0 = jnp.clip(xf, 0, W + 1)
                x1 = jnp.clip(xf + 1, 0, W + 1)
                y0 = jnp.clip(yf, 0, H + 1)
                y1 = jnp.clip(yf + 1, 0, H + 1)
                dxw = x1.astype(jnp.float32) - x
                dyw = y1.astype(jnp.float32) - y
                vx0 = (x0 >= 1) & (x0 <= W)
                vx1 = (x1 >= 1) & (x1 <= W)
                vy0 = (y0 >= 1) & (y0 <= H)
                vy1 = (y1 >= 1) & (y1 <= H)
                zero = jnp.zeros((16,), jnp.float32)
                wa = jnp.where(vx0 & vy0, dxw * dyw, zero)
                wb = jnp.where(vx0 & vy1, dxw * (1.0 - dyw), zero)
                wc = jnp.where(vx1 & vy0, (1.0 - dxw) * dyw, zero)
                wd = jnp.where(vx1 & vy1, (1.0 - dxw) * (1.0 - dyw), zero)
                xu0 = jnp.clip(x0 - 1, 0, W - 1)
                xu1 = jnp.clip(x1 - 1, 0, W - 1)
                yu0 = jnp.clip(y0 - 1, 0, H - 1) * W + bbase
                yu1 = jnp.clip(y1 - 1, 0, H - 1) * W + bbase
                idx4[S][0][s] = yu0 + xu0
                idx4[S][1][s] = yu1 + xu0
                idx4[S][2][s] = yu0 + xu1
                idx4[S][3][s] = yu1 + xu1
                wv4[S][0][s] = wa
                wv4[S][1][s] = wb
                wv4[S][2][s] = wc
                wv4[S][3][s] = wd

            fire_offsets(t + 2, S)
            for q in range(4):
                pltpu.async_copy(img_hbm.at[idx4[S][q]], bufs[S][q], semG[S])

        def wait_gathers(S):
            for q in range(4):
                pltpu.make_async_copy(img_hbm.at[idx4[S][q]], bufs[S][q], semG[S]).wait()

        def combine(S):
            ba, bb, bc, bd = bufs[S]
            wva, wvb, wvc, wvd = wv4[S]

            def body(i, carry2):
                s16 = pl.ds(i * 16, 16)
                wa16 = wva[s16]
                wb16 = wvb[s16]
                wc16 = wvc[s16]
                wd16 = wvd[s16]
                k0 = i * 16
                for k2 in range(16):
                    k = k0 + k2
                    wa = wa16[k2]
                    wb = wb16[k2]
                    wc = wc16[k2]
                    wd = wd16[k2]
                    for jj in range(C // 16):
                        s = pl.ds(jj * 16, 16)
                        outv[k, s] = (wa * ba[k, s] + wb * bb[k, s]
                                      + wc * bc[k, s] + wd * bd[k, s])
                return carry2

            lax.fori_loop(0, LG, body, 0)

        def store(t):
            p0, _ = p0_of(t)
            pltpu.async_copy(outv, out_hbm.at[pl.ds(p0, CHUNK), pl.ds(0, C)], semS)

        def drain_store(t):
            p0, _ = p0_of(t)
            pltpu.make_async_copy(outv, out_hbm.at[pl.ds(p0, CHUNK), pl.ds(0, C)], semS).wait()

        fire_offsets(0, 0)
        fire_offsets(1, 1)
        stage(0, 0)

        def body(t2, carry):
            t = 2 * t2
            stage(t + 1, 1)
            wait_gathers(0)

            @pl.when(t2 > 0)
            def _():
                drain_store(t - 1)

            combine(0)
            store(t)

            @pl.when(t2 < NCHUNK // 2 - 1)
            def _():
                stage(t + 2, 0)

            wait_gathers(1)
            drain_store(t)
            combine(1)
            store(t + 1)
            return carry

        lax.fori_loop(0, NCHUNK // 2, body, 0)
        drain_store(NCHUNK - 1)

    return _sc_body


def _sc_run(img128, dx, dy, q):
    kern = functools.partial(
        pl.kernel,
        mesh=plsc.VectorSubcoreMesh(core_axis_name="c", subcore_axis_name="s"),
        out_type=jax.ShapeDtypeStruct((NB, CP), jnp.float32),
        scratch_types=[
            pltpu.VMEM((2, CHUNK), jnp.float32),
            pltpu.VMEM((2, CHUNK), jnp.float32),
            [[pltpu.VMEM((CHUNK,), jnp.int32) for _ in range(4)]
             for _ in range(2)],
            [[pltpu.VMEM((CHUNK,), jnp.float32) for _ in range(4)]
             for _ in range(2)],
            [[pltpu.VMEM((CHUNK, CP), jnp.float32) for _ in range(4)]
             for _ in range(2)],
            pltpu.VMEM((CHUNK, C), jnp.float32),
            [pltpu.SemaphoreType.DMA for _ in range(2)],
            [pltpu.SemaphoreType.DMA for _ in range(2)],
            pltpu.SemaphoreType.DMA,
        ],
        compiler_params=pltpu.CompilerParams(use_tc_tiling_on_sc=False),
    )(_make_sc_body(q * BQ * HW))
    return kern(img128, dx, dy)


def kernel(image, offsets):
    imT = jnp.transpose(image, (0, 1, 3, 2))
    dx = offsets[..., 0].reshape(N)
    dy = offsets[..., 1].reshape(N)
    outs = []
    for q in range(NQ):
        img128 = _tc_prep(imT, q)
        outs.append(_sc_run(img128, dx, dy, q))
    outT = _tc_post(outs)
    return jnp.transpose(outT, (0, 1, 3, 2))


# speedup vs baseline: 3.0398x; 1.1720x over previous
import functools

import jax
import jax.numpy as jnp
from jax import lax
from jax.experimental import pallas as pl
from jax.experimental.pallas import tpu as pltpu
from jax.experimental.pallas import tpu_sc as plsc

B, H, W, C = 8, 224, 224, 96
CP = 128
HW = H * W
N = B * HW
NQ = 4
BQ = B // NQ
NB = BQ * HW
NW = 32
CHUNK = 112
NCHUNK = NB // (NW * CHUNK)
WPB = NW // BQ
RPW = H // WPB
LG = CHUNK // 16
ROWS = 8
HB = H // ROWS


def _floor(x):
    t = x.astype(jnp.int32)
    return jnp.where(x < t.astype(jnp.float32), t - 1, t)


def _prep_body(in_ref, out_ref):
    for h2 in range(ROWS):
        x = in_ref[0, h2]
        xp = jnp.pad(x, ((0, 32), (0, 32)))
        xt = xp.T
        out_ref[pl.ds(h2 * W, W), :] = xt[:W, :]


def _tc_prep(imT, q):
    return pl.pallas_call(
        _prep_body,
        grid=(BQ, HB),
        in_specs=[pl.BlockSpec((1, ROWS, C, W),
                               lambda b, h: (q * BQ + b, h, 0, 0))],
        out_specs=pl.BlockSpec((ROWS * W, CP), lambda b, h: (b * HB + h, 0)),
        out_shape=jax.ShapeDtypeStruct((NB, CP), jnp.float32),
    )(imT)


def _post_init_body(in_ref, out_ref):
    for h2 in range(ROWS):
        x = in_ref[pl.ds(h2 * W, W), :]
        xp = jnp.pad(x, ((0, 32), (0, 0)))
        xt = xp.T
        out_ref[0, h2] = xt[:C, :W]


def _post_upd_body(acc_ref, in_ref, out_ref):
    del acc_ref
    _post_init_body(in_ref, out_ref)


def _tc_post_q(acc, xq, k):
    xq_spec = pl.BlockSpec((ROWS * W, CP), lambda b, h: (b * HB + h, 0))
    out_spec = pl.BlockSpec((1, ROWS, C, W),
                            lambda b, h: (k * BQ + b, h, 0, 0))
    out_shape = jax.ShapeDtypeStruct((B, H, C, W), jnp.float32)
    if acc is None:
        return pl.pallas_call(
            _post_init_body,
            grid=(BQ, HB),
            in_specs=[xq_spec],
            out_specs=out_spec,
            out_shape=out_shape,
        )(xq)
    return pl.pallas_call(
        _post_upd_body,
        grid=(BQ, HB),
        in_specs=[pl.BlockSpec(memory_space=pl.ANY), xq_spec],
        out_specs=out_spec,
        out_shape=out_shape,
        input_output_aliases={0: 0},
    )(acc, xq)


def _make_sc_body(qoff):
    def _sc_body(img_hbm, dx_hbm, dy_hbm, out_hbm,
                 dxv, dyv, idx4, wv4, bufs, outv, semG, semO, semS):
        wid = lax.axis_index("s") * 2 + lax.axis_index("c")
        b = lax.shift_right_logical(wid, 4)
        r0 = (wid & (WPB - 1)) * RPW
        bbase = b * HW

        lanes = lax.iota(jnp.int32, 16)

        def p0_of(t):
            r = r0 + lax.shift_right_logical(t, 1)
            return bbase + r * W + (t & 1) * CHUNK, r

        def fire_offsets(t, S):
            @pl.when(t < NCHUNK)
            def _():
                p0, _ = p0_of(t)
                pltpu.async_copy(dx_hbm.at[pl.ds(p0 + qoff, CHUNK)], dxv.at[S], semO[S])
                pltpu.async_copy(dy_hbm.at[pl.ds(p0 + qoff, CHUNK)], dyv.at[S], semO[S])

        def stage(t, S):
            p0, r = p0_of(t)
            pltpu.make_async_copy(dx_hbm.at[pl.ds(p0 + qoff, CHUNK)], dxv.at[S], semO[S]).wait()
            pltpu.make_async_copy(dy_hbm.at[pl.ds(p0 + qoff, CHUNK)], dyv.at[S], semO[S]).wait()

            yb = r.astype(jnp.float32) + 1.0
            xb = float(S * CHUNK) + 1.0

            for i in range(LG):
                s = pl.ds(i * 16, 16)
                x = dxv[S, s] + ((lanes + i * 16).astype(jnp.float32) + xb)
                y = dyv[S, s] + yb
                x = jnp.clip(x, -8.0, 232.0)
                y = jnp.clip(y, -8.0, 232.0)
                xf = _floor(x)
                yf = _floor(y)
                x0 = jnp.clip(xf, 0, W + 1)
                x1 = jnp.clip(xf + 1, 0, W + 1)
                y0 = jnp.clip(yf, 0, H + 1)
                y1 = jnp.clip(yf + 1, 0, H + 1)
                dxw = x1.astype(jnp.float32) - x
                dyw = y1.astype(jnp.float32) - y
                vx0 = (x0 >= 1) & (x0 <= W)
                vx1 = (x1 >= 1) & (x1 <= W)
                vy0 = (y0 >= 1) & (y0 <= H)
                vy1 = (y1 >= 1) & (y1 <= H)
                zero = jnp.zeros((16,), jnp.float32)
                wa = jnp.where(vx0 & vy0, dxw * dyw, zero)
                wb = jnp.where(vx0 & vy1, dxw * (1.0 - dyw), zero)
                wc = jnp.where(vx1 & vy0, (1.0 - dxw) * dyw, zero)
                wd = jnp.where(vx1 & vy1, (1.0 - dxw) * (1.0 - dyw), zero)
                xu0 = jnp.clip(x0 - 1, 0, W - 1)
                xu1 = jnp.clip(x1 - 1, 0, W - 1)
                yu0 = jnp.clip(y0 - 1, 0, H - 1) * W + bbase
                yu1 = jnp.clip(y1 - 1, 0, H - 1) * W + bbase
                idx4[S][0][s] = yu0 + xu0
                idx4[S][1][s] = yu1 + xu0
                idx4[S][2][s] = yu0 + xu1
                idx4[S][3][s] = yu1 + xu1
                wv4[S][0][s] = wa
                wv4[S][1][s] = wb
                wv4[S][2][s] = wc
                wv4[S][3][s] = wd

            fire_offsets(t + 2, S)
            for q in range(4):
                pltpu.async_copy(img_hbm.at[idx4[S][q]], bufs[S][q], semG[S])

        def wait_gathers(S):
            for q in range(4):
                pltpu.make_async_copy(img_hbm.at[idx4[S][q]], bufs[S][q], semG[S]).wait()

        def combine(S):
            ba, bb, bc, bd = bufs[S]
            wva, wvb, wvc, wvd = wv4[S]

            def body(i, carry2):
                s16 = pl.ds(i * 16, 16)
                wa16 = wva[s16]
                wb16 = wvb[s16]
                wc16 = wvc[s16]
                wd16 = wvd[s16]
                k0 = i * 16
                for k2 in range(16):
                    k = k0 + k2
                    wa = wa16[k2]
                    wb = wb16[k2]
                    wc = wc16[k2]
                    wd = wd16[k2]
                    for jj in range(C // 16):
                        s = pl.ds(jj * 16, 16)
                        outv[k, s] = (wa * ba[k, s] + wb * bb[k, s]
                                      + wc * bc[k, s] + wd * bd[k, s])
                return carry2

            lax.fori_loop(0, LG, body, 0)

        def store(t):
            p0, _ = p0_of(t)
            pltpu.async_copy(outv, out_hbm.at[pl.ds(p0, CHUNK), pl.ds(0, C)], semS)

        def drain_store(t):
            p0, _ = p0_of(t)
            pltpu.make_async_copy(outv, out_hbm.at[pl.ds(p0, CHUNK), pl.ds(0, C)], semS).wait()

        fire_offsets(0, 0)
        fire_offsets(1, 1)
        stage(0, 0)

        def body(t2, carry):
            t = 2 * t2
            stage(t + 1, 1)
            wait_gathers(0)

            @pl.when(t2 > 0)
            def _():
                drain_store(t - 1)

            combine(0)
            store(t)

            @pl.when(t2 < NCHUNK // 2 - 1)
            def _():
                stage(t + 2, 0)

            wait_gathers(1)
            drain_store(t)
            combine(1)
            store(t + 1)
            return carry

        lax.fori_loop(0, NCHUNK // 2, body, 0)
        drain_store(NCHUNK - 1)

    return _sc_body


def _sc_run(img128, dx, dy, q):
    kern = functools.partial(
        pl.kernel,
        mesh=plsc.VectorSubcoreMesh(core_axis_name="c", subcore_axis_name="s"),
        out_type=jax.ShapeDtypeStruct((NB, CP), jnp.float32),
        scratch_types=[
            pltpu.VMEM((2, CHUNK), jnp.float32),
            pltpu.VMEM((2, CHUNK), jnp.float32),
            [[pltpu.VMEM((CHUNK,), jnp.int32) for _ in range(4)]
             for _ in range(2)],
            [[pltpu.VMEM((CHUNK,), jnp.float32) for _ in range(4)]
             for _ in range(2)],
            [[pltpu.VMEM((CHUNK, CP), jnp.float32) for _ in range(4)]
             for _ in range(2)],
            pltpu.VMEM((CHUNK, C), jnp.float32),
            [pltpu.SemaphoreType.DMA for _ in range(2)],
            [pltpu.SemaphoreType.DMA for _ in range(2)],
            pltpu.SemaphoreType.DMA,
        ],
        compiler_params=pltpu.CompilerParams(use_tc_tiling_on_sc=False),
    )(_make_sc_body(q * BQ * HW))
    return kern(img128, dx, dy)


def kernel(image, offsets):
    imT = jnp.transpose(image, (0, 1, 3, 2))
    dx = offsets[..., 0].reshape(N)
    dy = offsets[..., 1].reshape(N)
    acc = None
    for q in range(NQ):
        img128 = _tc_prep(imT, q)
        out_q = _sc_run(img128, dx, dy, q)
        acc = _tc_post_q(acc, out_q, q)
    return jnp.transpose(acc, (0, 1, 3, 2))


# speedup vs baseline: 4.1964x; 1.3805x over previous
import functools

import jax
import jax.numpy as jnp
from jax import lax
from jax.experimental import pallas as pl
from jax.experimental.pallas import tpu as pltpu
from jax.experimental.pallas import tpu_sc as plsc

B, H, W, C = 8, 224, 224, 96
CP = 128
HW = H * W
N = B * HW
NQ = 4
BQ = B // NQ
NB = BQ * HW
NW = 32
CHUNK = 112
NCHUNK = NB // (NW * CHUNK)
WPB = NW // BQ
RPW = H // WPB
LG = CHUNK // 16
ROWS = 8
HB = H // ROWS


def _floor(x):
    t = x.astype(jnp.int32)
    return jnp.where(x < t.astype(jnp.float32), t - 1, t)


def _prep_body(in_ref, out_ref):
    for h2 in range(ROWS):
        x = in_ref[0, h2]
        xp = jnp.pad(x, ((0, 32), (0, 32)))
        xt = xp.T
        out_ref[pl.ds(h2 * W, W), :] = xt[:W, :]


def _tc_prep(imT, q):
    return pl.pallas_call(
        _prep_body,
        grid=(BQ, HB),
        in_specs=[pl.BlockSpec((1, ROWS, C, W),
                               lambda b, h: (q * BQ + b, h, 0, 0))],
        out_specs=pl.BlockSpec((ROWS * W, CP), lambda b, h: (b * HB + h, 0)),
        out_shape=jax.ShapeDtypeStruct((NB, CP), jnp.float32),
    )(imT)


def _post_init_body(in_ref, out_ref):
    for h2 in range(ROWS):
        x = in_ref[pl.ds(h2 * W, W), :]
        xp = jnp.pad(x, ((0, 32), (0, 0)))
        xt = xp.T
        out_ref[0, h2] = xt[:C, :W]


def _post_upd_body(acc_ref, in_ref, out_ref):
    del acc_ref
    _post_init_body(in_ref, out_ref)


def _tc_post_q(acc, xq, k):
    xq_spec = pl.BlockSpec((ROWS * W, CP), lambda b, h: (b * HB + h, 0))
    out_spec = pl.BlockSpec((1, ROWS, C, W),
                            lambda b, h: (k * BQ + b, h, 0, 0))
    out_shape = jax.ShapeDtypeStruct((B, H, C, W), jnp.float32)
    if acc is None:
        return pl.pallas_call(
            _post_init_body,
            grid=(BQ, HB),
            in_specs=[xq_spec],
            out_specs=out_spec,
            out_shape=out_shape,
        )(xq)
    return pl.pallas_call(
        _post_upd_body,
        grid=(BQ, HB),
        in_specs=[pl.BlockSpec(memory_space=pl.ANY), xq_spec],
        out_specs=out_spec,
        out_shape=out_shape,
        input_output_aliases={0: 0},
    )(acc, xq)


def _make_sc_body(qoff):
    def _sc_body(img_hbm, dx_hbm, dy_hbm, out_hbm,
                 dxv, dyv, idx4, wv4, bufs, outv, semG, semO, semS):
        wid = lax.axis_index("s") * 2 + lax.axis_index("c")
        b = lax.shift_right_logical(wid, 4)
        r0 = (wid & (WPB - 1)) * RPW
        bbase = b * HW

        lanes = lax.iota(jnp.int32, 16)

        def p0_of(t):
            r = r0 + lax.shift_right_logical(t, 1)
            return bbase + r * W + (t & 1) * CHUNK, r

        def fire_offsets(t, S):
            @pl.when(t < NCHUNK)
            def _():
                p0, _ = p0_of(t)
                pltpu.async_copy(dx_hbm.at[pl.ds(p0 + qoff, CHUNK)], dxv.at[S], semO[S])
                pltpu.async_copy(dy_hbm.at[pl.ds(p0 + qoff, CHUNK)], dyv.at[S], semO[S])

        def stage(t, S):
            p0, r = p0_of(t)
            pltpu.make_async_copy(dx_hbm.at[pl.ds(p0 + qoff, CHUNK)], dxv.at[S], semO[S]).wait()
            pltpu.make_async_copy(dy_hbm.at[pl.ds(p0 + qoff, CHUNK)], dyv.at[S], semO[S]).wait()

            yb = r.astype(jnp.float32) + 1.0
            xb = float(S * CHUNK) + 1.0

            for i in range(LG):
                s = pl.ds(i * 16, 16)
                x = dxv[S, s] + ((lanes + i * 16).astype(jnp.float32) + xb)
                y = dyv[S, s] + yb
                x = jnp.clip(x, -8.0, 232.0)
                y = jnp.clip(y, -8.0, 232.0)
                xf = _floor(x)
                yf = _floor(y)
                x0 = jnp.clip(xf, 0, W + 1)
                x1 = jnp.clip(xf + 1, 0, W + 1)
                y0 = jnp.clip(yf, 0, H + 1)
                y1 = jnp.clip(yf + 1, 0, H + 1)
                dxw = x1.astype(jnp.float32) - x
                dyw = y1.astype(jnp.float32) - y
                vx0 = (x0 >= 1) & (x0 <= W)
                vx1 = (x1 >= 1) & (x1 <= W)
                vy0 = (y0 >= 1) & (y0 <= H)
                vy1 = (y1 >= 1) & (y1 <= H)
                zero = jnp.zeros((16,), jnp.float32)
                wa = jnp.where(vx0 & vy0, dxw * dyw, zero)
                wb = jnp.where(vx0 & vy1, dxw * (1.0 - dyw), zero)
                wc = jnp.where(vx1 & vy0, (1.0 - dxw) * dyw, zero)
                wd = jnp.where(vx1 & vy1, (1.0 - dxw) * (1.0 - dyw), zero)
                xu0 = jnp.clip(x0 - 1, 0, W - 1)
                xu1 = jnp.clip(x1 - 1, 0, W - 1)
                yu0 = jnp.clip(y0 - 1, 0, H - 1) * W + bbase
                yu1 = jnp.clip(y1 - 1, 0, H - 1) * W + bbase
                idx4[S][0][s] = yu0 + xu0
                idx4[S][1][s] = yu1 + xu0
                idx4[S][2][s] = yu0 + xu1
                idx4[S][3][s] = yu1 + xu1
                wv4[S][0][s] = wa
                wv4[S][1][s] = wb
                wv4[S][2][s] = wc
                wv4[S][3][s] = wd

            fire_offsets(t + 2, S)
            for q in range(4):
                pltpu.async_copy(img_hbm.at[idx4[S][q]], bufs[S][q], semG[S])

        def wait_gathers(S):
            for q in range(4):
                pltpu.make_async_copy(img_hbm.at[idx4[S][q]], bufs[S][q], semG[S]).wait()

        def combine(S):
            ba, bb, bc, bd = bufs[S]
            wva, wvb, wvc, wvd = wv4[S]

            @functools.partial(plsc.parallel_loop, 0, LG, unroll=2)
            def _(i):
                s16 = pl.ds(i * 16, 16)
                wa16 = wva[s16]
                wb16 = wvb[s16]
                wc16 = wvc[s16]
                wd16 = wvd[s16]
                k0 = i * 16
                for k2 in range(16):
                    k = k0 + k2
                    wa = wa16[k2]
                    wb = wb16[k2]
                    wc = wc16[k2]
                    wd = wd16[k2]
                    for jj in range(C // 16):
                        s = pl.ds(jj * 16, 16)
                        outv[k, s] = (wa * ba[k, s] + wb * bb[k, s]
                                      + wc * bc[k, s] + wd * bd[k, s])

        def store(t):
            p0, _ = p0_of(t)
            pltpu.async_copy(outv, out_hbm.at[pl.ds(p0, CHUNK), pl.ds(0, C)], semS)

        def drain_store(t):
            p0, _ = p0_of(t)
            pltpu.make_async_copy(outv, out_hbm.at[pl.ds(p0, CHUNK), pl.ds(0, C)], semS).wait()

        fire_offsets(0, 0)
        fire_offsets(1, 1)
        stage(0, 0)

        def body(t2, carry):
            t = 2 * t2
            stage(t + 1, 1)
            wait_gathers(0)

            @pl.when(t2 > 0)
            def _():
                drain_store(t - 1)

            combine(0)
            store(t)

            @pl.when(t2 < NCHUNK // 2 - 1)
            def _():
                stage(t + 2, 0)

            wait_gathers(1)
            drain_store(t)
            combine(1)
            store(t + 1)
            return carry

        lax.fori_loop(0, NCHUNK // 2, body, 0)
        drain_store(NCHUNK - 1)

    return _sc_body


def _sc_run(img128, dx, dy, q):
    kern = functools.partial(
        pl.kernel,
        mesh=plsc.VectorSubcoreMesh(core_axis_name="c", subcore_axis_name="s"),
        out_type=jax.ShapeDtypeStruct((NB, CP), jnp.float32),
        scratch_types=[
            pltpu.VMEM((2, CHUNK), jnp.float32),
            pltpu.VMEM((2, CHUNK), jnp.float32),
            [[pltpu.VMEM((CHUNK,), jnp.int32) for _ in range(4)]
             for _ in range(2)],
            [[pltpu.VMEM((CHUNK,), jnp.float32) for _ in range(4)]
             for _ in range(2)],
            [[pltpu.VMEM((CHUNK, CP), jnp.float32) for _ in range(4)]
             for _ in range(2)],
            pltpu.VMEM((CHUNK, C), jnp.float32),
            [pltpu.SemaphoreType.DMA for _ in range(2)],
            [pltpu.SemaphoreType.DMA for _ in range(2)],
            pltpu.SemaphoreType.DMA,
        ],
        compiler_params=pltpu.CompilerParams(use_tc_tiling_on_sc=False),
    )(_make_sc_body(q * BQ * HW))
    return kern(img128, dx, dy)


def kernel(image, offsets):
    imT = jnp.transpose(image, (0, 1, 3, 2))
    dx = offsets[..., 0].reshape(N)
    dy = offsets[..., 1].reshape(N)
    acc = None
    for q in range(NQ):
        img128 = _tc_prep(imT, q)
        out_q = _sc_run(img128, dx, dy, q)
        acc = _tc_post_q(acc, out_q, q)
    return jnp.transpose(acc, (0, 1, 3, 2))
